# Initial kernel scaffold; baseline (speedup 1.0000x reference)
import jax
import jax.numpy as jnp
from jax.experimental import pallas as pl


def kernel(token_ids, weights):
    raise NotImplementedError("write your pallas kernel here")


# speedup vs baseline: 4.9486x; 4.9486x over previous
import functools

import jax
import jax.numpy as jnp
from jax import lax
from jax.experimental import pallas as pl
from jax.experimental.pallas import tpu as pltpu
from jax.experimental.pallas import tpu_sc as plsc

NUM_ROWS = 1_000_000
DIM = 32
TOTAL = 16384 * 200
NUM_WORKERS = 32
PER_WORKER = TOTAL // NUM_WORKERS
CHUNK = 2048
NUM_CHUNKS = PER_WORKER // CHUNK


def _emb_body(idx_hbm, table_hbm, out_hbm, idx_v, rows_v, gsem, sem_out):
    wid = lax.axis_index("s") * 2 + lax.axis_index("c")
    base = wid * PER_WORKER

    @pl.loop(0, NUM_CHUNKS)
    def _chunk(i):
        off = base + i * CHUNK
        pltpu.sync_copy(idx_hbm.at[pl.ds(off, CHUNK)], idx_v)
        pltpu.async_copy(table_hbm.at[idx_v], rows_v, gsem).wait()
        pltpu.sync_copy(rows_v, out_hbm.at[pl.ds(off, CHUNK)])


@jax.jit
def _embedding_sc(token_ids_flat, weights):
    mesh = plsc.VectorSubcoreMesh(core_axis_name="c", subcore_axis_name="s")
    kfn = pl.kernel(
        _emb_body,
        out_type=jax.ShapeDtypeStruct((TOTAL, DIM), jnp.float32),
        mesh=mesh,
        scratch_types=[
            pltpu.VMEM((CHUNK,), jnp.int32),
            pltpu.VMEM((CHUNK, DIM), jnp.float32),
            pltpu.SemaphoreType.DMA,
            pltpu.SemaphoreType.DMA,
        ],
        compiler_params=pltpu.CompilerParams(use_tc_tiling_on_sc=False),
    )
    return kfn(token_ids_flat, weights)


def kernel(token_ids, weights):
    flat = token_ids.reshape(-1).astype(jnp.int32)
    out = _embedding_sc(flat, weights)
    return out.reshape(*token_ids.shape, DIM)


# speedup vs baseline: 4.9802x; 1.0064x over previous
import jax
import jax.numpy as jnp
from jax import lax
from jax.experimental import pallas as pl
from jax.experimental.pallas import tpu as pltpu
from jax.experimental.pallas import tpu_sc as plsc

NUM_ROWS = 1_000_000
DIM = 32
TOTAL = 16384 * 200
NUM_WORKERS = 32
PER_WORKER = TOTAL // NUM_WORKERS
CHUNK = 1600
NUM_CHUNKS = PER_WORKER // CHUNK
NBUF = 2


def _emb_body(idx_hbm, table_hbm, out_hbm, idx_v, rows_v, gsem):
    wid = lax.axis_index("s") * 2 + lax.axis_index("c")
    base = wid * PER_WORKER

    for b in range(NBUF):
        off = base + b * CHUNK
        pltpu.sync_copy(idx_hbm.at[pl.ds(off, CHUNK)], idx_v.at[b])
        pltpu.async_copy(table_hbm.at[idx_v.at[b]], rows_v.at[b], gsem.at[b])

    @pl.loop(0, NUM_CHUNKS // NBUF)
    def _grp(g):
        for b in range(NBUF):
            i = g * NBUF + b
            off = base + i * CHUNK
            pltpu.make_async_copy(
                table_hbm.at[idx_v.at[b]], rows_v.at[b], gsem.at[b]
            ).wait()
            pltpu.sync_copy(rows_v.at[b], out_hbm.at[pl.ds(off, CHUNK)])
            j = i + NBUF

            @pl.when(j < NUM_CHUNKS)
            def _refill():
                off2 = base + j * CHUNK
                pltpu.sync_copy(idx_hbm.at[pl.ds(off2, CHUNK)], idx_v.at[b])
                pltpu.async_copy(table_hbm.at[idx_v.at[b]], rows_v.at[b],
                                 gsem.at[b])


@jax.jit
def _embedding_sc(token_ids_flat, weights):
    mesh = plsc.VectorSubcoreMesh(core_axis_name="c", subcore_axis_name="s")
    kfn = pl.kernel(
        _emb_body,
        out_type=jax.ShapeDtypeStruct((TOTAL, DIM), jnp.float32),
        mesh=mesh,
        scratch_types=[
            pltpu.VMEM((NBUF, CHUNK), jnp.int32),
            pltpu.VMEM((NBUF, CHUNK, DIM), jnp.float32),
            pltpu.SemaphoreType.DMA((NBUF,)),
        ],
        compiler_params=pltpu.CompilerParams(use_tc_tiling_on_sc=False),
    )
    return kfn(token_ids_flat, weights)


def kernel(token_ids, weights):
    flat = token_ids.reshape(-1).astype(jnp.int32)
    out = _embedding_sc(flat, weights)
    return out.reshape(*token_ids.shape, DIM)


# speedup vs baseline: 4.9830x; 1.0006x over previous
import jax
import jax.numpy as jnp
from jax import lax
from jax.experimental import pallas as pl
from jax.experimental.pallas import tpu as pltpu
from jax.experimental.pallas import tpu_sc as plsc

NUM_ROWS = 1_000_000
DIM = 32
BATCH = 16384
SEQ = 200
NUM_WORKERS = 32
ROWS_PER_WORKER = BATCH // NUM_WORKERS
RCHUNK = 8
NUM_CHUNKS = ROWS_PER_WORKER // RCHUNK
NBUF = 2


def _emb_body(idx_hbm, table_hbm, out_hbm, idx_v, rows_v, gsem):
    wid = lax.axis_index("s") * 2 + lax.axis_index("c")
    row0 = wid * ROWS_PER_WORKER

    def _fire(b, r):
        pltpu.sync_copy(idx_hbm.at[pl.ds(r, RCHUNK)], idx_v.at[b])
        for k in range(RCHUNK):
            pltpu.async_copy(table_hbm.at[idx_v.at[b, k]], rows_v.at[b, k],
                             gsem.at[b])

    def _drain(b):
        for k in range(RCHUNK):
            pltpu.make_async_copy(
                table_hbm.at[idx_v.at[b, k]], rows_v.at[b, k], gsem.at[b]
            ).wait()

    for b in range(NBUF):
        _fire(b, row0 + b * RCHUNK)

    @pl.loop(0, NUM_CHUNKS // NBUF)
    def _grp(g):
        for b in range(NBUF):
            i = g * NBUF + b
            r = row0 + i * RCHUNK
            _drain(b)
            pltpu.sync_copy(rows_v.at[b], out_hbm.at[pl.ds(r, RCHUNK)])
            j = i + NBUF

            @pl.when(j < NUM_CHUNKS)
            def _refill():
                _fire(b, row0 + j * RCHUNK)


@jax.jit
def _embedding_sc(token_ids, weights):
    mesh = plsc.VectorSubcoreMesh(core_axis_name="c", subcore_axis_name="s")
    kfn = pl.kernel(
        _emb_body,
        out_type=jax.ShapeDtypeStruct((BATCH, SEQ, DIM), jnp.float32),
        mesh=mesh,
        scratch_types=[
            pltpu.VMEM((NBUF, RCHUNK, SEQ), jnp.int32),
            pltpu.VMEM((NBUF, RCHUNK, SEQ, DIM), jnp.float32),
            pltpu.SemaphoreType.DMA((NBUF,)),
        ],
        compiler_params=pltpu.CompilerParams(use_tc_tiling_on_sc=False),
    )
    return kfn(token_ids, weights)


def kernel(token_ids, weights):
    return _embedding_sc(token_ids, weights)
